# Initial kernel scaffold; baseline (speedup 1.0000x reference)
import jax
import jax.numpy as jnp
from jax.experimental import pallas as pl


def kernel(channel_indices, embedding):
    raise NotImplementedError("write your pallas kernel here")


# speedup vs baseline: 1.8860x; 1.8860x over previous
import functools

import jax
import jax.numpy as jnp
from jax import lax
from jax.experimental import pallas as pl
from jax.experimental.pallas import tpu as pltpu
from jax.experimental.pallas import tpu_sc as plsc

_B = 16384
_D = 3
_V = 4096
_L = 16

_info = plsc.get_sparse_core_info()
_NC = _info.num_cores
_NS = _info.num_subcores
_NW = _NC * _NS
_BPW = _B // _NW
_VECS = _BPW // _L


def _body(idx_hbm, tab_hbm, out_hbm, idx_v, tab_v, out_v, sem):
    wid = lax.axis_index("s") * _NC + lax.axis_index("c")
    base = wid * _BPW
    cp_idx = pltpu.async_copy(idx_hbm.at[pl.ds(base, _BPW)], idx_v, sem)
    cp_tab = pltpu.async_copy(tab_hbm, tab_v, sem)
    cp_idx.wait()
    cp_tab.wait()
    pos3 = lax.iota(jnp.int32, _L) * _D
    for k in range(_VECS):
        i16 = idx_v[pl.ds(_L * k, _L)]
        f16 = i16 * _D
        for c in range(_D):
            g = plsc.load_gather(tab_v, [f16 + c if c else f16])
            plsc.store_scatter(out_v, [pos3 + (_D * _L * k + c)], g)
    pltpu.sync_copy(out_v, out_hbm.at[pl.ds(base * _D, _BPW * _D)])


_gather_call = functools.partial(
    pl.kernel,
    mesh=plsc.VectorSubcoreMesh(core_axis_name="c", subcore_axis_name="s"),
    out_type=jax.ShapeDtypeStruct((_B * _D,), jnp.float32),
    scratch_types=[
        pltpu.VMEM((_BPW,), jnp.int32),
        pltpu.VMEM((_V * _D,), jnp.float32),
        pltpu.VMEM((_BPW * _D,), jnp.float32),
        pltpu.SemaphoreType.DMA,
    ],
    compiler_params=pltpu.CompilerParams(needs_layout_passes=False),
)(_body)


@jax.jit
def kernel(channel_indices, embedding):
    idx = channel_indices.astype(jnp.int32)
    tab = embedding.reshape(-1)
    out = _gather_call(idx, tab)
    return out.reshape(_B, _D)


# speedup vs baseline: 2.3144x; 1.2271x over previous
import functools

import jax
import jax.numpy as jnp
from jax import lax
from jax.experimental import pallas as pl
from jax.experimental.pallas import tpu as pltpu
from jax.experimental.pallas import tpu_sc as plsc

_B = 16384
_D = 3
_V = 4096
_L = 16

_info = plsc.get_sparse_core_info()
_NC = _info.num_cores
_NS = _info.num_subcores
_NW = _NC * _NS
_BPW = _B // _NW
_VECS = _BPW // _L


def _body(idx_hbm, tab_hbm, out_hbm, idx_v, tab_v, out_v, sem):
    wid = lax.axis_index("s") * _NC + lax.axis_index("c")
    base = wid * _BPW
    cp_idx = pltpu.async_copy(idx_hbm.at[pl.ds(base, _BPW)], idx_v, sem)
    cp_tab = pltpu.async_copy(tab_hbm, tab_v, sem)
    cp_idx.wait()
    cp_tab.wait()
    rows16 = lax.iota(jnp.int32, _L)
    for k in range(_VECS):
        i16 = idx_v[pl.ds(_L * k, _L)]
        f16 = i16 * _D
        r16 = rows16 + (_L * k)
        for c in range(_D):
            c16 = jnp.full((_L,), c, jnp.int32)
            g = plsc.load_gather(tab_v, [f16 + c if c else f16])
            plsc.store_scatter(out_v, [r16, c16], g)
    pltpu.sync_copy(out_v, out_hbm.at[pl.ds(base, _BPW), :])


_gather_call = functools.partial(
    pl.kernel,
    mesh=plsc.VectorSubcoreMesh(core_axis_name="c", subcore_axis_name="s"),
    out_type=jax.ShapeDtypeStruct((_B, _D), jnp.float32),
    scratch_types=[
        pltpu.VMEM((_BPW,), jnp.int32),
        pltpu.VMEM((_V * _D,), jnp.float32),
        pltpu.VMEM((_BPW, _D), jnp.float32),
        pltpu.SemaphoreType.DMA,
    ],
    compiler_params=pltpu.CompilerParams(needs_layout_passes=False),
)(_body)


@jax.jit
def kernel(channel_indices, embedding):
    return _gather_call(channel_indices.astype(jnp.int32), embedding.reshape(-1))


# speedup vs baseline: 3.2872x; 1.4203x over previous
import functools

import jax
import jax.numpy as jnp
from jax import lax
from jax.experimental import pallas as pl
from jax.experimental.pallas import tpu as pltpu
from jax.experimental.pallas import tpu_sc as plsc

_B = 16384
_D = 3
_V = 4096
_L = 16

_info = plsc.get_sparse_core_info()
_NC = _info.num_cores
_NS = _info.num_subcores
_NW = _NC * _NS
_BPW = _B // _NW
_VECS = _BPW // _L


def _body(idx_hbm, tab_hbm, out_hbm, idx_v, tab_v, out_v, sem):
    wid = lax.axis_index("s") * _NC + lax.axis_index("c")
    base = wid * _BPW
    cp_idx = pltpu.async_copy(idx_hbm.at[pl.ds(base, _BPW)], idx_v, sem)
    cp_tab = pltpu.async_copy(tab_hbm, tab_v, sem)
    cp_idx.wait()
    cp_tab.wait()
    rows16 = lax.iota(jnp.int32, _L)
    for k in range(_VECS):
        i16 = idx_v[pl.ds(_L * k, _L)]
        r16 = rows16 + (_L * k)
        for c in range(_D):
            c16 = jnp.full((_L,), c, jnp.int32)
            g = plsc.load_gather(tab_v, [i16 + (c * _V) if c else i16])
            plsc.store_scatter(out_v, [c16, r16], g)
    pltpu.sync_copy(out_v, out_hbm.at[:, pl.ds(base, _BPW)])


_gather_call = functools.partial(
    pl.kernel,
    mesh=plsc.VectorSubcoreMesh(core_axis_name="c", subcore_axis_name="s"),
    out_type=jax.ShapeDtypeStruct((_D, _B), jnp.float32),
    scratch_types=[
        pltpu.VMEM((_BPW,), jnp.int32),
        pltpu.VMEM((_V * _D,), jnp.float32),
        pltpu.VMEM((_D, _BPW), jnp.float32),
        pltpu.SemaphoreType.DMA,
    ],
    compiler_params=pltpu.CompilerParams(needs_layout_passes=False),
)(_body)


@jax.jit
def kernel(channel_indices, embedding):
    tab_t = embedding.T.reshape(-1)
    out_t = _gather_call(channel_indices.astype(jnp.int32), tab_t)
    return out_t.T
